# baseline (speedup 1.0000x reference)
import functools
import jax
import jax.numpy as jnp
from jax import lax
from jax.experimental import pallas as pl
from jax.experimental.pallas import tpu as pltpu
from jax.experimental.pallas import tpu_sc as plsc

_B = 16384
_NROWS = 1_000_000
_NT = 7813
_NFULL = 7812
_NC = 2
_NS = 16
_RPW = _B // _NS
_CHUNK = 128
_NCH = _RPW // _CHUNK
_WORDS = _NT * 1024
_TAIL0 = _NFULL * 128

_TPC = 31
_RPC = _TPC * 8


_NW = _NC * _NS
_SPW = _B // _NW
_NCHW = _SPW // _CHUNK


def _detile_body(ut_hbm, it_hbm, utail, itail, out_ulin, out_ilin,
                 ubuf, ibuf, tail_v, in_sems, out_sems):
    wid = lax.axis_index("s") * _NC + lax.axis_index("c")
    ntiles = jnp.where(wid < 4, 245, 244)
    t0 = jnp.minimum(wid, 4) * 245 + jnp.maximum(wid - 4, 0) * 244

    def drain2(_):
        pltpu.make_async_copy(ut_hbm.at[:, pl.ds(0, 128)],
                              out_ulin.at[pl.ds(0, 8)], in_sems.at[0]).wait()
        pltpu.make_async_copy(it_hbm.at[:, pl.ds(0, 128)],
                              out_ilin.at[pl.ds(0, 8)], in_sems.at[0]).wait()

    def body(i, _):
        t = t0 + i
        pltpu.async_copy(ut_hbm.at[:, pl.ds(t * 128, 128)],
                         out_ulin.at[pl.ds(t * 8, 8)], in_sems.at[0])
        pltpu.async_copy(it_hbm.at[:, pl.ds(t * 128, 128)],
                         out_ilin.at[pl.ds(t * 8, 8)], in_sems.at[0])
        @pl.when(i >= 8)
        def _():
            drain2(None)
        return 0

    lax.fori_loop(0, ntiles, body, 0)

    def tail_drain(i, _):
        @pl.when(i < jnp.minimum(ntiles, 8))
        def _():
            drain2(None)
        return 0

    lax.fori_loop(0, 8, tail_drain, 0)

    @pl.when(wid == _NW - 1)
    def _():
        pltpu.sync_copy(utail, tail_v)
        pltpu.sync_copy(tail_v, out_ulin.at[pl.ds(_NFULL * 8, 8)])
        pltpu.sync_copy(itail, tail_v)
        pltpu.sync_copy(tail_v, out_ilin.at[pl.ds(_NFULL * 8, 8)])


_detile = functools.partial(
    pl.kernel,
    mesh=plsc.VectorSubcoreMesh(core_axis_name="c", subcore_axis_name="s"),
    out_type=[jax.ShapeDtypeStruct((_NT * 8, 128), jnp.float32),
              jax.ShapeDtypeStruct((_NT * 8, 128), jnp.float32)],
    scratch_types=[
        pltpu.VMEM((2, _RPC, 128), jnp.float32),
        pltpu.VMEM((2, _RPC, 128), jnp.float32),
        pltpu.VMEM((8, 128), jnp.float32),
        pltpu.SemaphoreType.DMA((2,)),
        pltpu.SemaphoreType.DMA((2,)),
    ],
    compiler_params=pltpu.CompilerParams(use_tc_tiling_on_sc=True),
)(_detile_body)


def _addrs(idx_v, addr_v, row0, ch):
    for g in range(_CHUNK // 16):
        off = ch * _CHUNK + g * 16
        r = idx_v[pl.ds(off, 16)]
        tmask = r >= _TAIL0
        a0 = jnp.where(tmask,
                       _NFULL * 1024 + (r - _TAIL0) * 8,
                       (r >> 7) * 1024 + (r & 127))
        step = jnp.where(tmask, 1, 128)
        for e in range(8):
            addr_v[row0 + e, pl.ds(off, 16)] = a0 + step * e


def _gather_body(uidx, iidx, utab_lin, itab_lin, out_ux, out_ix,
                 uidx_v, iidx_v, addr_v, gbuf_v, sem):
    wid = lax.axis_index("s") * _NC + lax.axis_index("c")
    base = wid * _SPW
    pltpu.sync_copy(uidx.at[pl.ds(base, _SPW)], uidx_v)
    pltpu.sync_copy(iidx.at[pl.ds(base, _SPW)], iidx_v)

    for ch in range(_NCHW):
        _addrs(uidx_v, addr_v, 0, ch)
        _addrs(iidx_v, addr_v, 8, ch)
    copies = []
    for ch in range(_NCHW):
        for e in range(8):
            copies.append(pltpu.async_copy(
                utab_lin.at[addr_v.at[e, pl.ds(ch * _CHUNK, _CHUNK)]],
                gbuf_v.at[e, pl.ds(ch * _CHUNK, _CHUNK)], sem))
        for e in range(8):
            copies.append(pltpu.async_copy(
                itab_lin.at[addr_v.at[8 + e, pl.ds(ch * _CHUNK, _CHUNK)]],
                gbuf_v.at[8 + e, pl.ds(ch * _CHUNK, _CHUNK)], sem))
    for c in copies:
        c.wait()

    pltpu.sync_copy(gbuf_v.at[pl.ds(0, 8)], out_ux.at[:, pl.ds(base, _SPW)])
    pltpu.sync_copy(gbuf_v.at[pl.ds(8, 8)], out_ix.at[:, pl.ds(base, _SPW)])


_sc_gather = functools.partial(
    pl.kernel,
    mesh=plsc.VectorSubcoreMesh(core_axis_name="c", subcore_axis_name="s"),
    out_type=[jax.ShapeDtypeStruct((8, _B), jnp.float32),
              jax.ShapeDtypeStruct((8, _B), jnp.float32)],
    scratch_types=[
        pltpu.VMEM((_SPW,), jnp.int32),
        pltpu.VMEM((_SPW,), jnp.int32),
        pltpu.VMEM((16, _SPW), jnp.int32),
        pltpu.VMEM((16, _SPW), jnp.float32),
        pltpu.SemaphoreType.DMA,
    ],
    compiler_params=pltpu.CompilerParams(use_tc_tiling_on_sc=True),
)(_gather_body)


def _mlp_body(u_ref, i_ref, w1u_ref, w1i_ref, b1_ref, w2_ref, b2_ref,
              w3_ref, b3_ref, wf_ref, bf_ref, out_ref):
    dn = (((0,), (0,)), ((), ()))
    h = (lax.dot_general(w1u_ref[...], u_ref[...], dn)
         + lax.dot_general(w1i_ref[...], i_ref[...], dn) + b1_ref[...])
    h = jnp.maximum(h, 0.0)
    h = jnp.maximum(lax.dot_general(w2_ref[...], h, dn) + b2_ref[...], 0.0)
    h = jnp.maximum(lax.dot_general(w3_ref[...], h, dn) + b3_ref[...], 0.0)
    out_ref[...] = jax.nn.sigmoid(
        lax.dot_general(wf_ref[...], h, dn) + bf_ref[...])


_MLP_BLK = 4096


def _mlp(u_t, i_t, w1u, w1i, b1, w2, b2, w3, b3, wf, bf):
    grid = _B // _MLP_BLK
    rep = lambda shape: pl.BlockSpec(shape, lambda g: (0,) * len(shape))
    return pl.pallas_call(
        _mlp_body,
        grid=(grid,),
        in_specs=[
            pl.BlockSpec((8, _MLP_BLK), lambda g: (0, g)),
            pl.BlockSpec((8, _MLP_BLK), lambda g: (0, g)),
            rep((8, 32)), rep((8, 32)), rep((32, 1)),
            rep((32, 64)), rep((64, 1)),
            rep((64, 32)), rep((32, 1)),
            rep((32, 1)), rep((1, 1)),
        ],
        out_specs=pl.BlockSpec((1, _MLP_BLK), lambda g: (0, g)),
        out_shape=jax.ShapeDtypeStruct((1, _B), jnp.float32),
    )(u_t, i_t, w1u, w1i, b1, w2, b2, w3, b3, wf, bf)


@jax.jit
def kernel(user_input, item_input, user_table, item_table,
           W1, b1, W2, b2, W3, b3, Wf, bf):
    utail = jnp.pad(user_table[_TAIL0:].reshape(512), (0, 512)).reshape(8, 128)
    itail = jnp.pad(item_table[_TAIL0:].reshape(512), (0, 512)).reshape(8, 128)
    ut_lin, it_lin = _detile(user_table.T, item_table.T, utail, itail)
    u_t, i_t = _sc_gather(
        user_input.astype(jnp.int32), item_input.astype(jnp.int32),
        ut_lin.reshape(_WORDS), it_lin.reshape(_WORDS))
    pred_t = _mlp(u_t, i_t, W1[:8], W1[8:], b1.reshape(-1, 1),
                  W2, b2.reshape(-1, 1), W3, b3.reshape(-1, 1),
                  Wf, bf.reshape(1, 1))
    return pred_t.reshape(_B, 1)


# speedup vs baseline: 22.7990x; 22.7990x over previous
import functools
import jax
import jax.numpy as jnp
from jax import lax
from jax.experimental import pallas as pl
from jax.experimental.pallas import tpu as pltpu
from jax.experimental.pallas import tpu_sc as plsc

_B = 16384
_NROWS = 1_000_000
_NT = 7813
_NFULL = 7812
_NC = 2
_NS = 16
_RPW = _B // _NS
_CHUNK = 128
_NCH = _RPW // _CHUNK
_WORDS = _NT * 1024
_TAIL0 = _NFULL * 128

_TPC = 31
_RPC = _TPC * 8


_NW = _NC * _NS
_SPW = _B // _NW
_NCHW = _SPW // _CHUNK


def _detile_body(ut_hbm, it_hbm, utail, itail, out_ulin, out_ilin,
                 ubuf, ibuf, tail_v, in_sems, out_sems):
    wid = lax.axis_index("s") * _NC + lax.axis_index("c")
    nfull = jnp.where(wid < 28, 8, 7)
    cb = jnp.minimum(wid, 28) * 8 + jnp.maximum(wid - 28, 0) * 7

    def fire_in(k):
        slot = lax.rem(k, 2)
        col0 = (cb + k) * (_TPC * 128)
        for m in range(_TPC):
            pltpu.async_copy(ut_hbm.at[:, pl.ds(col0 + m * 128, 128)],
                             ubuf.at[slot, pl.ds(m * 8, 8)], in_sems.at[slot])
            pltpu.async_copy(it_hbm.at[:, pl.ds(col0 + m * 128, 128)],
                             ibuf.at[slot, pl.ds(m * 8, 8)], in_sems.at[slot])

    @pl.when(nfull > 0)
    def _():
        fire_in(jnp.int32(0))

    def body(k, _):
        slot = lax.rem(k, 2)
        nslot = lax.rem(k + 1, 2)
        @pl.when((k >= 1) & (k + 1 < nfull))
        def _():
            pltpu.make_async_copy(ubuf.at[nslot], out_ulin.at[pl.ds(0, _RPC)],
                                  out_sems.at[nslot]).wait()
            pltpu.make_async_copy(ibuf.at[nslot], out_ilin.at[pl.ds(0, _RPC)],
                                  out_sems.at[nslot]).wait()
        @pl.when(k + 1 < nfull)
        def _():
            fire_in(k + 1)
        for _i in range(2 * _TPC):
            pltpu.make_async_copy(ut_hbm.at[:, pl.ds(0, 128)],
                                  ubuf.at[slot, pl.ds(0, 8)],
                                  in_sems.at[slot]).wait()
        row0 = (cb + k) * _RPC
        pltpu.async_copy(ubuf.at[slot], out_ulin.at[pl.ds(row0, _RPC)],
                         out_sems.at[slot])
        pltpu.async_copy(ibuf.at[slot], out_ilin.at[pl.ds(row0, _RPC)],
                         out_sems.at[slot])
        return 0

    lax.fori_loop(0, nfull, body, 0)

    def final_drain(k, _):
        slot = lax.rem(k, 2)
        @pl.when(k + 2 >= nfull)
        def _():
            pltpu.make_async_copy(ubuf.at[slot], out_ulin.at[pl.ds(0, _RPC)],
                                  out_sems.at[slot]).wait()
            pltpu.make_async_copy(ibuf.at[slot], out_ilin.at[pl.ds(0, _RPC)],
                                  out_sems.at[slot]).wait()
        return 0

    lax.fori_loop(0, nfull, final_drain, 0)

    @pl.when(wid == _NW - 1)
    def _():
        pltpu.sync_copy(utail, tail_v)
        pltpu.sync_copy(tail_v, out_ulin.at[pl.ds(_NFULL * 8, 8)])
        pltpu.sync_copy(itail, tail_v)
        pltpu.sync_copy(tail_v, out_ilin.at[pl.ds(_NFULL * 8, 8)])


_detile = functools.partial(
    pl.kernel,
    mesh=plsc.VectorSubcoreMesh(core_axis_name="c", subcore_axis_name="s"),
    out_type=[jax.ShapeDtypeStruct((_NT * 8, 128), jnp.float32),
              jax.ShapeDtypeStruct((_NT * 8, 128), jnp.float32)],
    scratch_types=[
        pltpu.VMEM((2, _RPC, 128), jnp.float32),
        pltpu.VMEM((2, _RPC, 128), jnp.float32),
        pltpu.VMEM((8, 128), jnp.float32),
        pltpu.SemaphoreType.DMA((2,)),
        pltpu.SemaphoreType.DMA((2,)),
    ],
    compiler_params=pltpu.CompilerParams(use_tc_tiling_on_sc=True),
)(_detile_body)


def _addrs(idx_v, addr_v, row0, ch):
    for g in range(_CHUNK // 16):
        off = ch * _CHUNK + g * 16
        r = idx_v[pl.ds(off, 16)]
        tmask = r >= _TAIL0
        a0 = jnp.where(tmask,
                       _NFULL * 1024 + (r - _TAIL0) * 8,
                       (r >> 7) * 1024 + (r & 127))
        step = jnp.where(tmask, 1, 128)
        for e in range(8):
            addr_v[row0 + e, pl.ds(off, 16)] = a0 + step * e


def _gather_body(uidx, iidx, utab_lin, itab_lin, out_ux, out_ix,
                 uidx_v, iidx_v, addr_v, gbuf_v, sem):
    wid = lax.axis_index("s") * _NC + lax.axis_index("c")
    base = wid * _SPW
    pltpu.sync_copy(uidx.at[pl.ds(base, _SPW)], uidx_v)
    pltpu.sync_copy(iidx.at[pl.ds(base, _SPW)], iidx_v)

    for ch in range(_NCHW):
        _addrs(uidx_v, addr_v, 0, ch)
        _addrs(iidx_v, addr_v, 8, ch)
    copies = []
    for ch in range(_NCHW):
        for e in range(8):
            copies.append(pltpu.async_copy(
                utab_lin.at[addr_v.at[e, pl.ds(ch * _CHUNK, _CHUNK)]],
                gbuf_v.at[e, pl.ds(ch * _CHUNK, _CHUNK)], sem))
        for e in range(8):
            copies.append(pltpu.async_copy(
                itab_lin.at[addr_v.at[8 + e, pl.ds(ch * _CHUNK, _CHUNK)]],
                gbuf_v.at[8 + e, pl.ds(ch * _CHUNK, _CHUNK)], sem))
    for c in copies:
        c.wait()

    pltpu.sync_copy(gbuf_v.at[pl.ds(0, 8)], out_ux.at[:, pl.ds(base, _SPW)])
    pltpu.sync_copy(gbuf_v.at[pl.ds(8, 8)], out_ix.at[:, pl.ds(base, _SPW)])


_sc_gather = functools.partial(
    pl.kernel,
    mesh=plsc.VectorSubcoreMesh(core_axis_name="c", subcore_axis_name="s"),
    out_type=[jax.ShapeDtypeStruct((8, _B), jnp.float32),
              jax.ShapeDtypeStruct((8, _B), jnp.float32)],
    scratch_types=[
        pltpu.VMEM((_SPW,), jnp.int32),
        pltpu.VMEM((_SPW,), jnp.int32),
        pltpu.VMEM((16, _SPW), jnp.int32),
        pltpu.VMEM((16, _SPW), jnp.float32),
        pltpu.SemaphoreType.DMA,
    ],
    compiler_params=pltpu.CompilerParams(use_tc_tiling_on_sc=True),
)(_gather_body)


def _mlp_body(u_ref, i_ref, w1u_ref, w1i_ref, b1_ref, w2_ref, b2_ref,
              w3_ref, b3_ref, wf_ref, bf_ref, out_ref):
    dn = (((0,), (0,)), ((), ()))
    h = (lax.dot_general(w1u_ref[...], u_ref[...], dn)
         + lax.dot_general(w1i_ref[...], i_ref[...], dn) + b1_ref[...])
    h = jnp.maximum(h, 0.0)
    h = jnp.maximum(lax.dot_general(w2_ref[...], h, dn) + b2_ref[...], 0.0)
    h = jnp.maximum(lax.dot_general(w3_ref[...], h, dn) + b3_ref[...], 0.0)
    out_ref[...] = jax.nn.sigmoid(
        lax.dot_general(wf_ref[...], h, dn) + bf_ref[...])


_MLP_BLK = 4096


def _mlp(u_t, i_t, w1u, w1i, b1, w2, b2, w3, b3, wf, bf):
    grid = _B // _MLP_BLK
    rep = lambda shape: pl.BlockSpec(shape, lambda g: (0,) * len(shape))
    return pl.pallas_call(
        _mlp_body,
        grid=(grid,),
        in_specs=[
            pl.BlockSpec((8, _MLP_BLK), lambda g: (0, g)),
            pl.BlockSpec((8, _MLP_BLK), lambda g: (0, g)),
            rep((8, 32)), rep((8, 32)), rep((32, 1)),
            rep((32, 64)), rep((64, 1)),
            rep((64, 32)), rep((32, 1)),
            rep((32, 1)), rep((1, 1)),
        ],
        out_specs=pl.BlockSpec((1, _MLP_BLK), lambda g: (0, g)),
        out_shape=jax.ShapeDtypeStruct((1, _B), jnp.float32),
    )(u_t, i_t, w1u, w1i, b1, w2, b2, w3, b3, wf, bf)


@jax.jit
def kernel(user_input, item_input, user_table, item_table,
           W1, b1, W2, b2, W3, b3, Wf, bf):
    utail = jnp.pad(user_table[_TAIL0:].reshape(512), (0, 512)).reshape(8, 128)
    itail = jnp.pad(item_table[_TAIL0:].reshape(512), (0, 512)).reshape(8, 128)
    ut_lin, it_lin = _detile(user_table.T, item_table.T, utail, itail)
    u_t, i_t = _sc_gather(
        user_input.astype(jnp.int32), item_input.astype(jnp.int32),
        ut_lin.reshape(_WORDS), it_lin.reshape(_WORDS))
    pred_t = _mlp(u_t, i_t, W1[:8], W1[8:], b1.reshape(-1, 1),
                  W2, b2.reshape(-1, 1), W3, b3.reshape(-1, 1),
                  Wf, bf.reshape(1, 1))
    return pred_t.reshape(_B, 1)
